# baseline (speedup 1.0000x reference)
import jax
import jax.numpy as jnp
import numpy as np
from jax.experimental import pallas as pl
from jax.experimental.pallas import tpu as pltpu

_B, _N = 2, 2048
_K1, _K2, _K3 = 21, 3, 5
_C = 10
_TILE = 256
_F = _C * _K2 * (_K3 - 1)


def _rowmin_idx(d, iota):
    m = jnp.min(d, axis=1, keepdims=True)
    idx = jnp.min(jnp.where(d <= m, iota, _N), axis=1, keepdims=True)
    return m, idx


def _gather3(idx, iota, X, Y, Z):
    sel = iota == idx
    gx = jnp.sum(jnp.where(sel, X, 0.0), axis=1, keepdims=True)
    gy = jnp.sum(jnp.where(sel, Y, 0.0), axis=1, keepdims=True)
    gz = jnp.sum(jnp.where(sel, Z, 0.0), axis=1, keepdims=True)
    return gx, gy, gz


def _tile_body(cb_ref, xt_ref, k1_ref, b1_ref, k2_ref, b2_ref, k3_ref,
               b3_ref, out_ref):
    X = cb_ref[0, 0:1, :]
    Y = cb_ref[0, 1:2, :]
    Z = cb_ref[0, 2:3, :]
    xt = xt_ref[0]
    qx = xt[:, 0:1]
    qy = xt[:, 1:2]
    qz = xt[:, 2:3]
    iota = jax.lax.broadcasted_iota(jnp.int32, (1, _N), 1)
    lane4 = jax.lax.broadcasted_iota(jnp.int32, (1, _K3 - 1), 1)
    inf = jnp.float32(jnp.inf)

    x2all = X * X + Y * Y + Z * Z
    q2 = qx * qx + qy * qy + qz * qz

    def bf(a):
        return a.astype(jnp.bfloat16).astype(jnp.float32)

    Xb, Yb, Zb = bf(X), bf(Y), bf(Z)
    dot = bf(qx) * Xb + bf(qy) * Yb + bf(qz) * Zb
    d1 = (q2 + x2all) - 2.0 * dot

    def _extract(k, carry):
        dw, e0, e1 = carry
        _, idx = _rowmin_idx(dw, iota)
        e0 = jnp.where(k == 0, idx, e0)
        e1 = jnp.where(k == 1, idx, e1)
        return jnp.where(iota == idx, inf, dw), e0, e1

    zeroi = jnp.zeros((_TILE, 1), jnp.int32)
    dwork, e0, e1 = jax.lax.fori_loop(0, _K1, _extract, (d1, zeroi, zeroi))
    group_mask = (dwork == inf) & (iota != e0)

    c0x, c0y, c0z = _gather3(e1, iota, X, Y, Z)
    dist0 = (X - c0x) ** 2 + (Y - c0y) ** 2 + (Z - c0z) ** 2
    v = jnp.where(group_mask, dist0, -inf)
    m = jnp.max(v, axis=1, keepdims=True)
    f1 = jnp.min(jnp.where(v >= m, iota, _N), axis=1, keepdims=True)
    c1x, c1y, c1z = _gather3(f1, iota, X, Y, Z)
    dist1 = (X - c1x) ** 2 + (Y - c1y) ** 2 + (Z - c1z) ** 2
    v = jnp.where(group_mask, jnp.minimum(dist0, dist1), -inf)
    m = jnp.max(v, axis=1, keepdims=True)
    f2 = jnp.min(jnp.where(v >= m, iota, _N), axis=1, keepdims=True)
    c2x, c2y, c2z = _gather3(f2, iota, X, Y, Z)

    centroids = [(c0x, c0y, c0z), (c1x, c1y, c1z), (c2x, c2y, c2z)]

    nbr = []
    for (cx, cy, cz) in centroids:
        c2s = cx * cx + cy * cy + cz * cz
        cdot = bf(cx) * Xb + bf(cy) * Yb + bf(cz) * Zb
        d2 = (c2s + x2all) - 2.0 * cdot
        zero4 = jnp.zeros((_TILE, _K3 - 1), jnp.float32)

        def _step(k, carry):
            d2c, ax, ay, az = carry
            _, idx = _rowmin_idx(d2c, iota)
            gx, gy, gz = _gather3(idx, iota, X, Y, Z)
            oh = jnp.where(lane4 == (k - 1), jnp.float32(1.0),
                           jnp.float32(0.0))
            return (jnp.where(iota == idx, inf, d2c),
                    ax + gx * oh, ay + gy * oh, az + gz * oh)

        _, ax, ay, az = jax.lax.fori_loop(
            0, _K3, _step, (d2, zero4, zero4, zero4))
        nbr.append((ax - cx, ay - cy, az - cz))

    g = []
    for j in range(_K3 - 1):
        gx = jnp.concatenate([nbr[i][0][:, j:j + 1] for i in range(_K2)],
                             axis=1)
        gy = jnp.concatenate([nbr[i][1][:, j:j + 1] for i in range(_K2)],
                             axis=1)
        gz = jnp.concatenate([nbr[i][2][:, j:j + 1] for i in range(_K2)],
                             axis=1)
        g.append((jnp.arctan2(gy, gx), gx, gy, gz))

    for (a, b) in ((0, 1), (2, 3), (0, 2), (1, 3), (1, 2)):
        swap = g[a][0] > g[b][0]
        lo = tuple(jnp.where(swap, vb, va) for va, vb in zip(g[a], g[b]))
        hi = tuple(jnp.where(swap, va, vb) for va, vb in zip(g[a], g[b]))
        g[a], g[b] = lo, hi

    pm = None
    feats = []
    for j in range(_K3 - 1):
        _, sx, sy, sz = g[j]
        _, rx, ry, rz = g[(j + 1) % (_K3 - 1)]
        nx = sy * rz - sz * ry
        ny = sz * rx - sx * rz
        nz = sx * ry - sy * rx
        nrm = jnp.maximum(jnp.sqrt(nx * nx + ny * ny + nz * nz), 1e-12)
        ux, uy, uz = nx / nrm, ny / nrm, nz / nrm
        if j == 0:
            pm = jnp.where(ux > 0, jnp.float32(1.0), jnp.float32(-1.0))
        cgx = (sx + rx) / 3.0
        cgy = (sy + ry) / 3.0
        cgz = (sz + rz) / 3.0
        rho = jnp.sqrt(cgx * cgx + cgy * cgy + cgz * cgz)
        t = jnp.clip(cgz / jnp.maximum(rho, 1e-12), -1.0, 1.0)
        theta = jnp.arctan2(jnp.sqrt(jnp.maximum(1.0 - t * t, 0.0)),
                            t) / np.pi
        phi = jnp.arctan2(cgy, cgx) / (2.0 * np.pi) + 0.5
        feats.append([cgx, cgy, cgz, rho, theta, phi, ux, uy, uz])
    cols = []
    for j in range(_K3 - 1):
        cgx, cgy, cgz, rho, theta, phi, ux, uy, uz = feats[j]
        nxs, nys, nzs = ux * pm, uy * pm, uz * pm
        pos = (nxs * cgx + nys * cgy + nzs * cgz) / np.sqrt(3.0)
        cols.append([cgx, cgy, cgz, rho, theta, phi, nxs, nys, nzs, pos])

    F = jnp.concatenate(
        [jnp.concatenate([cols[j][f] for j in range(_K3 - 1)], axis=1)
         for f in range(_C)], axis=1)

    Fh = jnp.maximum(jnp.dot(F, k1_ref[:, :],
                             preferred_element_type=jnp.float32)
                     + b1_ref[:, :], 0.0)
    Fh = jnp.maximum(jnp.dot(Fh, k2_ref[:, :],
                             preferred_element_type=jnp.float32)
                     + b2_ref[:, :], 0.0)
    Fh = jnp.dot(Fh, k3_ref[:, :],
                 preferred_element_type=jnp.float32) + b3_ref[:, :]

    outs = []
    for o in range(_C):
        p = Fh[:, 12 * o:12 * o + 12]
        q = jnp.maximum(jnp.maximum(p[:, 0:3], p[:, 3:6]),
                        jnp.maximum(p[:, 6:9], p[:, 9:12]))
        outs.append((q[:, 0:1] + q[:, 1:2] + q[:, 2:3]) / 3.0)
    out_ref[0] = jnp.concatenate(outs, axis=1)


def kernel(center, W1, g1, be1, m1, v1, W2, b2, g2, be2, m2, v2, W3, b3):
    xyzT = jnp.transpose(center, (0, 2, 1))
    inv1 = g1 / jnp.sqrt(v1 + 1e-5)
    Wf1 = W1 * inv1[:, None]
    bf1 = be1 - m1 * inv1
    inv2 = g2 / jnp.sqrt(v2 + 1e-5)
    Wf2 = W2 * inv2[:, None]
    bf2 = (b2 - m2) * inv2 + be2
    eye = jnp.eye(_K2 * (_K3 - 1), dtype=jnp.float32)
    K1m = jnp.kron(Wf1.T, eye)
    K2m = jnp.kron(Wf2.T, eye)
    K3m = jnp.kron(W3.T, eye)
    B1 = jnp.repeat(bf1, _K2 * (_K3 - 1)).reshape(1, _F)
    B2 = jnp.repeat(bf2, _K2 * (_K3 - 1)).reshape(1, _F)
    B3 = jnp.repeat(b3, _K2 * (_K3 - 1)).reshape(1, _F)

    grid = (_B, _N // _TILE)
    out = pl.pallas_call(
        _tile_body,
        grid=grid,
        in_specs=[
            pl.BlockSpec((1, 3, _N), lambda b, t: (b, 0, 0)),
            pl.BlockSpec((1, _TILE, 3), lambda b, t: (b, t, 0)),
            pl.BlockSpec((_F, _F), lambda b, t: (0, 0)),
            pl.BlockSpec((1, _F), lambda b, t: (0, 0)),
            pl.BlockSpec((_F, _F), lambda b, t: (0, 0)),
            pl.BlockSpec((1, _F), lambda b, t: (0, 0)),
            pl.BlockSpec((_F, _F), lambda b, t: (0, 0)),
            pl.BlockSpec((1, _F), lambda b, t: (0, 0)),
        ],
        out_specs=pl.BlockSpec((1, _TILE, _C), lambda b, t: (b, t, 0)),
        out_shape=jax.ShapeDtypeStruct((_B, _N, _C), jnp.float32),
        compiler_params=pltpu.CompilerParams(
            dimension_semantics=("parallel", "parallel")),
    )(center, xyzT, K1m, B1, K2m, B2, K3m, B3)
    return jnp.transpose(out, (0, 2, 1))


# speedup vs baseline: 1.0377x; 1.0377x over previous
import jax
import jax.numpy as jnp
import numpy as np
from jax.experimental import pallas as pl
from jax.experimental.pallas import tpu as pltpu

_B, _N = 2, 2048
_K1, _K2, _K3 = 21, 3, 5
_C = 10
_TILE = 256
_F = _C * _K2 * (_K3 - 1)


def _rowmin_idx(d, iota):
    m = jnp.min(d, axis=1, keepdims=True)
    idx = jnp.min(jnp.where(d <= m, iota, _N), axis=1, keepdims=True)
    return m, idx


def _gather3(idx, iota, X, Y, Z):
    sel = iota == idx
    gx = jnp.sum(jnp.where(sel, X, 0.0), axis=1, keepdims=True)
    gy = jnp.sum(jnp.where(sel, Y, 0.0), axis=1, keepdims=True)
    gz = jnp.sum(jnp.where(sel, Z, 0.0), axis=1, keepdims=True)
    return gx, gy, gz


def _tile_body(cb_ref, xt_ref, k1_ref, b1_ref, k2_ref, b2_ref, k3_ref,
               b3_ref, out_ref):
    X = cb_ref[0, 0:1, :]
    Y = cb_ref[0, 1:2, :]
    Z = cb_ref[0, 2:3, :]
    xt = xt_ref[0]
    qx = xt[:, 0:1]
    qy = xt[:, 1:2]
    qz = xt[:, 2:3]
    iota = jax.lax.broadcasted_iota(jnp.int32, (1, _N), 1)
    lane4 = jax.lax.broadcasted_iota(jnp.int32, (1, _K3 - 1), 1)
    inf = jnp.float32(jnp.inf)

    x2all = X * X + Y * Y + Z * Z
    q2 = qx * qx + qy * qy + qz * qz

    xyzb = cb_ref[0].astype(jnp.bfloat16)
    dot = jnp.dot(xt.astype(jnp.bfloat16), xyzb,
                  preferred_element_type=jnp.float32)
    d1 = (q2 + x2all) - 2.0 * dot

    def _extract(k, carry):
        dw, e0, e1 = carry
        _, idx = _rowmin_idx(dw, iota)
        e0 = jnp.where(k == 0, idx, e0)
        e1 = jnp.where(k == 1, idx, e1)
        return jnp.where(iota == idx, inf, dw), e0, e1

    zeroi = jnp.zeros((_TILE, 1), jnp.int32)
    dwork, e0, e1 = jax.lax.fori_loop(0, _K1, _extract, (d1, zeroi, zeroi))
    group_mask = (dwork == inf) & (iota != e0)

    c0x, c0y, c0z = _gather3(e1, iota, X, Y, Z)
    dist0 = (X - c0x) ** 2 + (Y - c0y) ** 2 + (Z - c0z) ** 2
    v = jnp.where(group_mask, dist0, -inf)
    m = jnp.max(v, axis=1, keepdims=True)
    f1 = jnp.min(jnp.where(v >= m, iota, _N), axis=1, keepdims=True)
    c1x, c1y, c1z = _gather3(f1, iota, X, Y, Z)
    dist1 = (X - c1x) ** 2 + (Y - c1y) ** 2 + (Z - c1z) ** 2
    v = jnp.where(group_mask, jnp.minimum(dist0, dist1), -inf)
    m = jnp.max(v, axis=1, keepdims=True)
    f2 = jnp.min(jnp.where(v >= m, iota, _N), axis=1, keepdims=True)
    c2x, c2y, c2z = _gather3(f2, iota, X, Y, Z)

    centroids = [(c0x, c0y, c0z), (c1x, c1y, c1z), (c2x, c2y, c2z)]

    nbr = []
    for (cx, cy, cz) in centroids:
        c2s = cx * cx + cy * cy + cz * cz
        cmat = jnp.concatenate([cx, cy, cz], axis=1).astype(jnp.bfloat16)
        cdot = jnp.dot(cmat, xyzb, preferred_element_type=jnp.float32)
        d2 = (c2s + x2all) - 2.0 * cdot
        zero4 = jnp.zeros((_TILE, _K3 - 1), jnp.float32)

        def _step(k, carry):
            d2c, ax, ay, az = carry
            _, idx = _rowmin_idx(d2c, iota)
            gx, gy, gz = _gather3(idx, iota, X, Y, Z)
            oh = jnp.where(lane4 == (k - 1), jnp.float32(1.0),
                           jnp.float32(0.0))
            return (jnp.where(iota == idx, inf, d2c),
                    ax + gx * oh, ay + gy * oh, az + gz * oh)

        _, ax, ay, az = jax.lax.fori_loop(
            0, _K3, _step, (d2, zero4, zero4, zero4))
        nbr.append((ax - cx, ay - cy, az - cz))

    g = []
    for j in range(_K3 - 1):
        gx = jnp.concatenate([nbr[i][0][:, j:j + 1] for i in range(_K2)],
                             axis=1)
        gy = jnp.concatenate([nbr[i][1][:, j:j + 1] for i in range(_K2)],
                             axis=1)
        gz = jnp.concatenate([nbr[i][2][:, j:j + 1] for i in range(_K2)],
                             axis=1)
        g.append((jnp.arctan2(gy, gx), gx, gy, gz))

    for (a, b) in ((0, 1), (2, 3), (0, 2), (1, 3), (1, 2)):
        swap = g[a][0] > g[b][0]
        lo = tuple(jnp.where(swap, vb, va) for va, vb in zip(g[a], g[b]))
        hi = tuple(jnp.where(swap, va, vb) for va, vb in zip(g[a], g[b]))
        g[a], g[b] = lo, hi

    pm = None
    feats = []
    for j in range(_K3 - 1):
        _, sx, sy, sz = g[j]
        _, rx, ry, rz = g[(j + 1) % (_K3 - 1)]
        nx = sy * rz - sz * ry
        ny = sz * rx - sx * rz
        nz = sx * ry - sy * rx
        nrm = jnp.maximum(jnp.sqrt(nx * nx + ny * ny + nz * nz), 1e-12)
        ux, uy, uz = nx / nrm, ny / nrm, nz / nrm
        if j == 0:
            pm = jnp.where(ux > 0, jnp.float32(1.0), jnp.float32(-1.0))
        cgx = (sx + rx) / 3.0
        cgy = (sy + ry) / 3.0
        cgz = (sz + rz) / 3.0
        rho = jnp.sqrt(cgx * cgx + cgy * cgy + cgz * cgz)
        t = jnp.clip(cgz / jnp.maximum(rho, 1e-12), -1.0, 1.0)
        theta = jnp.arctan2(jnp.sqrt(jnp.maximum(1.0 - t * t, 0.0)),
                            t) / np.pi
        phi = jnp.arctan2(cgy, cgx) / (2.0 * np.pi) + 0.5
        feats.append([cgx, cgy, cgz, rho, theta, phi, ux, uy, uz])
    cols = []
    for j in range(_K3 - 1):
        cgx, cgy, cgz, rho, theta, phi, ux, uy, uz = feats[j]
        nxs, nys, nzs = ux * pm, uy * pm, uz * pm
        pos = (nxs * cgx + nys * cgy + nzs * cgz) / np.sqrt(3.0)
        cols.append([cgx, cgy, cgz, rho, theta, phi, nxs, nys, nzs, pos])

    F = jnp.concatenate(
        [jnp.concatenate([cols[j][f] for j in range(_K3 - 1)], axis=1)
         for f in range(_C)], axis=1)

    Fh = jnp.maximum(jnp.dot(F, k1_ref[:, :],
                             preferred_element_type=jnp.float32)
                     + b1_ref[:, :], 0.0)
    Fh = jnp.maximum(jnp.dot(Fh, k2_ref[:, :],
                             preferred_element_type=jnp.float32)
                     + b2_ref[:, :], 0.0)
    Fh = jnp.dot(Fh, k3_ref[:, :],
                 preferred_element_type=jnp.float32) + b3_ref[:, :]

    outs = []
    for o in range(_C):
        p = Fh[:, 12 * o:12 * o + 12]
        q = jnp.maximum(jnp.maximum(p[:, 0:3], p[:, 3:6]),
                        jnp.maximum(p[:, 6:9], p[:, 9:12]))
        outs.append((q[:, 0:1] + q[:, 1:2] + q[:, 2:3]) / 3.0)
    out_ref[0] = jnp.concatenate(outs, axis=1)


def kernel(center, W1, g1, be1, m1, v1, W2, b2, g2, be2, m2, v2, W3, b3):
    xyzT = jnp.transpose(center, (0, 2, 1))
    inv1 = g1 / jnp.sqrt(v1 + 1e-5)
    Wf1 = W1 * inv1[:, None]
    bf1 = be1 - m1 * inv1
    inv2 = g2 / jnp.sqrt(v2 + 1e-5)
    Wf2 = W2 * inv2[:, None]
    bf2 = (b2 - m2) * inv2 + be2
    eye = jnp.eye(_K2 * (_K3 - 1), dtype=jnp.float32)
    K1m = jnp.kron(Wf1.T, eye)
    K2m = jnp.kron(Wf2.T, eye)
    K3m = jnp.kron(W3.T, eye)
    B1 = jnp.repeat(bf1, _K2 * (_K3 - 1)).reshape(1, _F)
    B2 = jnp.repeat(bf2, _K2 * (_K3 - 1)).reshape(1, _F)
    B3 = jnp.repeat(b3, _K2 * (_K3 - 1)).reshape(1, _F)

    grid = (_B, _N // _TILE)
    out = pl.pallas_call(
        _tile_body,
        grid=grid,
        in_specs=[
            pl.BlockSpec((1, 3, _N), lambda b, t: (b, 0, 0)),
            pl.BlockSpec((1, _TILE, 3), lambda b, t: (b, t, 0)),
            pl.BlockSpec((_F, _F), lambda b, t: (0, 0)),
            pl.BlockSpec((1, _F), lambda b, t: (0, 0)),
            pl.BlockSpec((_F, _F), lambda b, t: (0, 0)),
            pl.BlockSpec((1, _F), lambda b, t: (0, 0)),
            pl.BlockSpec((_F, _F), lambda b, t: (0, 0)),
            pl.BlockSpec((1, _F), lambda b, t: (0, 0)),
        ],
        out_specs=pl.BlockSpec((1, _TILE, _C), lambda b, t: (b, t, 0)),
        out_shape=jax.ShapeDtypeStruct((_B, _N, _C), jnp.float32),
        compiler_params=pltpu.CompilerParams(
            dimension_semantics=("parallel", "parallel")),
    )(center, xyzT, K1m, B1, K2m, B2, K3m, B3)
    return jnp.transpose(out, (0, 2, 1))
